# baseline (speedup 1.0000x reference)
import jax
import jax.numpy as jnp
from jax import lax
from jax.experimental import pallas as pl
from jax.experimental.pallas import tpu as pltpu
from jax.experimental.pallas import tpu_sc as plsc

N = 10000
D = 256
OUT = 256
HEADS = 4
NNZ = 160000

NC = 2
NS = 16
LANES = 16
HC = OUT // NC
CHUNK = 64
NNZP = 163840
EPW = NNZP // NS
NCH = EPW // CHUNK
RPW = 624


BLK = 1000
BLK_E = 16384


def _tc_body(u_ref, i_ref, atta_ref, attw_ref, w1_ref, w2_ref, bsum_ref,
             vals_ref, z_ref, base_ref, vrep_ref):
    pid = pl.program_id(0)
    x = jnp.where(pid < (N // BLK) // 2, u_ref[...], i_ref[...])
    scores = jnp.dot(x, atta_ref[...].T,
                     preferred_element_type=jnp.float32)
    scores = jnp.where(scores >= 0, scores, 0.2 * scores)
    head_live = lax.broadcasted_iota(jnp.int32, (BLK, 8), 1) < HEADS
    scores = jnp.where(head_live, scores, -1e30)
    scores = scores - jnp.max(scores, axis=1, keepdims=True)
    e = jnp.exp(scores)
    alpha = e / jnp.sum(e, axis=1, keepdims=True)

    f = jnp.zeros((BLK, D), jnp.float32)
    for h in range(HEADS):
        f = f + alpha[:, h:h + 1] * jnp.dot(
            x, attw_ref[h], preferred_element_type=jnp.float32)

    y1 = jnp.dot(f, w1_ref[...], preferred_element_type=jnp.float32)
    z = y1 + jnp.dot(f * f, w2_ref[...], preferred_element_type=jnp.float32)
    base = y1 + bsum_ref[...]
    z_ref[0] = z[:, :HC]
    z_ref[1] = z[:, HC:]
    base_ref[0] = base[:, :HC]
    base_ref[1] = base[:, HC:]
    vrep_ref[...] = jnp.broadcast_to(vals_ref[...], (BLK_E, LANES))


def _tc_dense(user, item, att_a_pad, att_W, w1, w2, bsum, vals1):
    grid = N // BLK
    half = grid // 2
    return pl.pallas_call(
        _tc_body,
        grid=(grid,),
        in_specs=[
            pl.BlockSpec((BLK, D), lambda i: (jnp.minimum(i, half - 1), 0)),
            pl.BlockSpec((BLK, D),
                         lambda i: (jnp.maximum(i - half, 0), 0)),
            pl.BlockSpec((8, D), lambda i: (0, 0)),
            pl.BlockSpec((HEADS, D, D), lambda i: (0, 0, 0)),
            pl.BlockSpec((D, OUT), lambda i: (0, 0)),
            pl.BlockSpec((D, OUT), lambda i: (0, 0)),
            pl.BlockSpec((1, OUT), lambda i: (0, 0)),
            pl.BlockSpec((BLK_E, 1), lambda i: (i, 0)),
        ],
        out_specs=[
            pl.BlockSpec((NC, BLK, HC), lambda i: (0, i, 0)),
            pl.BlockSpec((NC, BLK, HC), lambda i: (0, i, 0)),
            pl.BlockSpec((BLK_E, LANES), lambda i: (i, 0)),
        ],
        out_shape=[
            jax.ShapeDtypeStruct((NC, N, HC), jnp.float32),
            jax.ShapeDtypeStruct((NC, N, HC), jnp.float32),
            jax.ShapeDtypeStruct((NNZP, LANES), jnp.float32),
        ],
    )(user, item, att_a_pad, att_W, w1, w2, bsum, vals1)


def _sc_body(z_hbm, base_hbm, rows_hbm, cols_hbm, vrep_hbm, out_hbm,
             cols_v, vb0, vb1, gb0, gb1, sb0, sb1,
             si0, si1, si2, si3, acc,
             gs0, gs1, vs0, vs1, ss0, ss1):
    c = lax.axis_index("c")
    s = lax.axis_index("s")
    vbuf = (vb0, vb1)
    gbuf = (gb0, gb1)
    sbuf = (sb0, sb1)
    sidx = (si0, si1, si2, si3)
    gsem = (gs0, gs1)
    vsem = (vs0, vs1)
    ssem = (ss0, ss1)

    r0 = s * RPW
    pltpu.sync_copy(base_hbm.at[pl.ds(c * N + r0, RPW)], acc.at[pl.ds(r0, RPW)])

    @pl.when(s == NS - 1)
    def _():
        pltpu.sync_copy(base_hbm.at[pl.ds(c * N + NS * RPW, N - NS * RPW)],
                        acc.at[pl.ds(NS * RPW, N - NS * RPW)])

    eb = s * EPW
    pltpu.sync_copy(cols_hbm.at[pl.ds(c * NNZP + eb, EPW)], cols_v)

    plsc.subcore_barrier()

    def start(j, b, b4):
        pltpu.async_copy(
            vrep_hbm.at[pl.ds((eb + j * CHUNK) * LANES, CHUNK * LANES)],
            vbuf[b], vsem[b])
        pltpu.async_copy(rows_hbm.at[pl.ds(eb + j * CHUNK, CHUNK)],
                         sidx[b4], gsem[b])
        pltpu.async_copy(z_hbm.at[cols_v.at[pl.ds(j * CHUNK, CHUNK)]],
                         gbuf[b], gsem[b])

    def wait(j, b, b4):
        pltpu.make_async_copy(
            vrep_hbm.at[pl.ds((eb + j * CHUNK) * LANES, CHUNK * LANES)],
            vbuf[b], vsem[b]).wait()
        pltpu.make_async_copy(rows_hbm.at[pl.ds(eb + j * CHUNK, CHUNK)],
                              sidx[b4], gsem[b]).wait()
        pltpu.make_async_copy(z_hbm.at[cols_v.at[pl.ds(j * CHUNK, CHUNK)]],
                              gbuf[b], gsem[b]).wait()

    def wait_scatter(b):
        pltpu.make_async_copy(sbuf[b], acc.at[sidx[0]], ssem[b]).wait()

    for b in range(2):
        start(b, b, b)

    def quad(jj, carry):
        for b4 in range(4):
            j = 4 * jj + b4
            b = b4 % 2
            wait(j, b, b4)

            @pl.when(j >= 2)
            def _():
                wait_scatter(b)

            def scale(g, cy):
                for u in range(8):
                    e = g * 8 + u
                    vspl = vbuf[b][pl.ds(e * LANES, LANES)]
                    for d in range(HC // LANES):
                        sbuf[b][e, pl.ds(d * LANES, LANES)] = (
                            gbuf[b][e, pl.ds(d * LANES, LANES)] * vspl)
                return cy

            lax.fori_loop(0, CHUNK // 8, scale, 0)
            pltpu.async_copy(sbuf[b], acc.at[sidx[b4]], ssem[b], add=True)

            @pl.when(j + 2 < NCH)
            def _():
                start(j + 2, b, (b4 + 2) % 4)
        return carry

    lax.fori_loop(0, NCH // 4, quad, 0)

    for b in range(2):
        wait_scatter(b)

    plsc.subcore_barrier()

    pltpu.sync_copy(acc.at[pl.ds(r0, RPW)],
                    out_hbm.at[pl.ds(r0, RPW), pl.ds(c * HC, HC)])

    @pl.when(s == NS - 1)
    def _():
        pltpu.sync_copy(acc.at[pl.ds(NS * RPW, N - NS * RPW)],
                        out_hbm.at[pl.ds(NS * RPW, N - NS * RPW),
                                   pl.ds(c * HC, HC)])


def _sc_spmm(z2, base2, rows, cols2, vrep):
    mesh = plsc.VectorSubcoreMesh(core_axis_name="c", subcore_axis_name="s",
                                  num_cores=NC, num_subcores=NS)
    return pl.kernel(
        _sc_body,
        out_type=jax.ShapeDtypeStruct((N, OUT), jnp.float32),
        mesh=mesh,
        scratch_types=[
            pltpu.VMEM((EPW,), jnp.int32),
            pltpu.VMEM((CHUNK * LANES,), jnp.float32),
            pltpu.VMEM((CHUNK * LANES,), jnp.float32),
            pltpu.VMEM((CHUNK, HC), jnp.float32),
            pltpu.VMEM((CHUNK, HC), jnp.float32),
            pltpu.VMEM((CHUNK, HC), jnp.float32),
            pltpu.VMEM((CHUNK, HC), jnp.float32),
            pltpu.VMEM((CHUNK,), jnp.int32),
            pltpu.VMEM((CHUNK,), jnp.int32),
            pltpu.VMEM((CHUNK,), jnp.int32),
            pltpu.VMEM((CHUNK,), jnp.int32),
            pltpu.VMEM_SHARED((N, HC), jnp.float32),
            pltpu.SemaphoreType.DMA,
            pltpu.SemaphoreType.DMA,
            pltpu.SemaphoreType.DMA,
            pltpu.SemaphoreType.DMA,
            pltpu.SemaphoreType.DMA,
            pltpu.SemaphoreType.DMA,
        ],
    )(z2, base2, rows, cols2, vrep)


def kernel(userFeatures, itemFeatures, att_a, att_W, affine1_W, affine1_b,
           affine2_W, affine2_b, lap_vals, lap_rows, lap_cols):
    att_a_pad = jnp.zeros((8, D), jnp.float32).at[:HEADS].set(att_a)
    bsum = (affine1_b + affine2_b).reshape(1, OUT)
    pad = NNZP - NNZ
    vals1 = jnp.concatenate(
        [lap_vals, jnp.zeros((pad,), jnp.float32)]).reshape(NNZP, 1)
    z2, base2, vrep2 = _tc_dense(userFeatures, itemFeatures, att_a_pad,
                                 att_W, affine1_W, affine2_W, bsum, vals1)
    z2 = z2.reshape(NC * N, HC)
    base2 = base2.reshape(NC * N, HC)
    vrep = vrep2.reshape(NNZP * LANES)

    rows = jnp.concatenate(
        [lap_rows.astype(jnp.int32), jnp.zeros((pad,), jnp.int32)])
    cols = jnp.concatenate(
        [lap_cols.astype(jnp.int32), jnp.zeros((pad,), jnp.int32)])
    cols2 = jnp.concatenate([cols, cols + N])

    return _sc_spmm(z2, base2, rows, cols2, vrep)


# speedup vs baseline: 1.0369x; 1.0369x over previous
import jax
import jax.numpy as jnp
from jax import lax
from jax.experimental import pallas as pl
from jax.experimental.pallas import tpu as pltpu
from jax.experimental.pallas import tpu_sc as plsc

N = 10000
D = 256
OUT = 256
HEADS = 4
NNZ = 160000

NC = 2
NS = 16
LANES = 16
HC = OUT // NC
CHUNK = 64
NNZP = 163840
EPW = NNZP // NS
NCH = EPW // CHUNK
RPW = 624


BLK = 1000


def _tc_body(x_ref, atta_ref, attw_ref, w1_ref, w2_ref, bsum_ref,
             z_ref, base_ref):
    x = x_ref[...]
    scores = jnp.dot(x, atta_ref[...].T,
                     preferred_element_type=jnp.float32)
    scores = jnp.where(scores >= 0, scores, 0.2 * scores)
    head_live = lax.broadcasted_iota(jnp.int32, (BLK, 8), 1) < HEADS
    scores = jnp.where(head_live, scores, -1e30)
    scores = scores - jnp.max(scores, axis=1, keepdims=True)
    e = jnp.exp(scores)
    alpha = e / jnp.sum(e, axis=1, keepdims=True)

    f = jnp.zeros((BLK, D), jnp.float32)
    for h in range(HEADS):
        f = f + alpha[:, h:h + 1] * jnp.dot(
            x, attw_ref[h], preferred_element_type=jnp.float32)

    y1 = jnp.dot(f, w1_ref[...], preferred_element_type=jnp.float32)
    z = y1 + jnp.dot(f * f, w2_ref[...], preferred_element_type=jnp.float32)
    base = y1 + bsum_ref[...]
    z_ref[0] = z[:, :HC]
    z_ref[1] = z[:, HC:]
    base_ref[0] = base[:, :HC]
    base_ref[1] = base[:, HC:]


def _tc_dense(feats, att_a_pad, att_W, w1, w2, bsum):
    grid = N // BLK
    return pl.pallas_call(
        _tc_body,
        grid=(grid,),
        in_specs=[
            pl.BlockSpec((BLK, D), lambda i: (i, 0)),
            pl.BlockSpec((8, D), lambda i: (0, 0)),
            pl.BlockSpec((HEADS, D, D), lambda i: (0, 0, 0)),
            pl.BlockSpec((D, OUT), lambda i: (0, 0)),
            pl.BlockSpec((D, OUT), lambda i: (0, 0)),
            pl.BlockSpec((1, OUT), lambda i: (0, 0)),
        ],
        out_specs=[
            pl.BlockSpec((NC, BLK, HC), lambda i: (0, i, 0)),
            pl.BlockSpec((NC, BLK, HC), lambda i: (0, i, 0)),
        ],
        out_shape=[
            jax.ShapeDtypeStruct((NC, N, HC), jnp.float32),
            jax.ShapeDtypeStruct((NC, N, HC), jnp.float32),
        ],
    )(feats, att_a_pad, att_W, w1, w2, bsum)


def _sc_body(z_hbm, base_hbm, rows_hbm, cols_hbm, vrep_hbm, out_hbm,
             cols_v, vb0, vb1, gb0, gb1, sb0, sb1,
             si0, si1, si2, si3, acc,
             gs0, gs1, vs0, vs1, ss0, ss1):
    c = lax.axis_index("c")
    s = lax.axis_index("s")
    vbuf = (vb0, vb1)
    gbuf = (gb0, gb1)
    sbuf = (sb0, sb1)
    sidx = (si0, si1, si2, si3)
    gsem = (gs0, gs1)
    vsem = (vs0, vs1)
    ssem = (ss0, ss1)

    r0 = s * RPW
    pltpu.sync_copy(base_hbm.at[pl.ds(c * N + r0, RPW)], acc.at[pl.ds(r0, RPW)])

    @pl.when(s == NS - 1)
    def _():
        pltpu.sync_copy(base_hbm.at[pl.ds(c * N + NS * RPW, N - NS * RPW)],
                        acc.at[pl.ds(NS * RPW, N - NS * RPW)])

    eb = s * EPW
    pltpu.sync_copy(cols_hbm.at[pl.ds(c * NNZP + eb, EPW)], cols_v)

    plsc.subcore_barrier()

    def start(j, b, b4):
        pltpu.async_copy(
            vrep_hbm.at[pl.ds((eb + j * CHUNK) * LANES, CHUNK * LANES)],
            vbuf[b], vsem[b])
        pltpu.async_copy(rows_hbm.at[pl.ds(eb + j * CHUNK, CHUNK)],
                         sidx[b4], gsem[b])
        pltpu.async_copy(z_hbm.at[cols_v.at[pl.ds(j * CHUNK, CHUNK)]],
                         gbuf[b], gsem[b])

    def wait(j, b, b4):
        pltpu.make_async_copy(
            vrep_hbm.at[pl.ds((eb + j * CHUNK) * LANES, CHUNK * LANES)],
            vbuf[b], vsem[b]).wait()
        pltpu.make_async_copy(rows_hbm.at[pl.ds(eb + j * CHUNK, CHUNK)],
                              sidx[b4], gsem[b]).wait()
        pltpu.make_async_copy(z_hbm.at[cols_v.at[pl.ds(j * CHUNK, CHUNK)]],
                              gbuf[b], gsem[b]).wait()

    def wait_scatter(b):
        pltpu.make_async_copy(sbuf[b], acc.at[sidx[0]], ssem[b]).wait()

    for b in range(2):
        start(b, b, b)

    def quad(jj, carry):
        for b4 in range(4):
            j = 4 * jj + b4
            b = b4 % 2
            wait(j, b, b4)

            @pl.when(j >= 2)
            def _():
                wait_scatter(b)

            def scale(g, cy):
                for u in range(8):
                    e = g * 8 + u
                    vspl = vbuf[b][pl.ds(e * LANES, LANES)]
                    for d in range(HC // LANES):
                        sbuf[b][e, pl.ds(d * LANES, LANES)] = (
                            gbuf[b][e, pl.ds(d * LANES, LANES)] * vspl)
                return cy

            lax.fori_loop(0, CHUNK // 8, scale, 0)
            pltpu.async_copy(sbuf[b], acc.at[sidx[b4]], ssem[b], add=True)

            @pl.when(j + 2 < NCH)
            def _():
                start(j + 2, b, (b4 + 2) % 4)
        return carry

    lax.fori_loop(0, NCH // 4, quad, 0)

    for b in range(2):
        wait_scatter(b)

    plsc.subcore_barrier()

    pltpu.sync_copy(acc.at[pl.ds(r0, RPW)],
                    out_hbm.at[pl.ds(r0, RPW), pl.ds(c * HC, HC)])

    @pl.when(s == NS - 1)
    def _():
        pltpu.sync_copy(acc.at[pl.ds(NS * RPW, N - NS * RPW)],
                        out_hbm.at[pl.ds(NS * RPW, N - NS * RPW),
                                   pl.ds(c * HC, HC)])


def _sc_spmm(z2, base2, rows, cols2, vrep):
    mesh = plsc.VectorSubcoreMesh(core_axis_name="c", subcore_axis_name="s",
                                  num_cores=NC, num_subcores=NS)
    return pl.kernel(
        _sc_body,
        out_type=jax.ShapeDtypeStruct((N, OUT), jnp.float32),
        mesh=mesh,
        scratch_types=[
            pltpu.VMEM((EPW,), jnp.int32),
            pltpu.VMEM((CHUNK * LANES,), jnp.float32),
            pltpu.VMEM((CHUNK * LANES,), jnp.float32),
            pltpu.VMEM((CHUNK, HC), jnp.float32),
            pltpu.VMEM((CHUNK, HC), jnp.float32),
            pltpu.VMEM((CHUNK, HC), jnp.float32),
            pltpu.VMEM((CHUNK, HC), jnp.float32),
            pltpu.VMEM((CHUNK,), jnp.int32),
            pltpu.VMEM((CHUNK,), jnp.int32),
            pltpu.VMEM((CHUNK,), jnp.int32),
            pltpu.VMEM((CHUNK,), jnp.int32),
            pltpu.VMEM_SHARED((N, HC), jnp.float32),
            pltpu.SemaphoreType.DMA,
            pltpu.SemaphoreType.DMA,
            pltpu.SemaphoreType.DMA,
            pltpu.SemaphoreType.DMA,
            pltpu.SemaphoreType.DMA,
            pltpu.SemaphoreType.DMA,
        ],
    )(z2, base2, rows, cols2, vrep)


def kernel(userFeatures, itemFeatures, att_a, att_W, affine1_W, affine1_b,
           affine2_W, affine2_b, lap_vals, lap_rows, lap_cols):
    feats = jnp.concatenate([userFeatures, itemFeatures], axis=0)
    att_a_pad = jnp.zeros((8, D), jnp.float32).at[:HEADS].set(att_a)
    bsum = (affine1_b + affine2_b).reshape(1, OUT)
    z2, base2 = _tc_dense(feats, att_a_pad, att_W, affine1_W, affine2_W,
                          bsum)
    z2 = z2.reshape(NC * N, HC)
    base2 = base2.reshape(NC * N, HC)

    pad = NNZP - NNZ
    rows = jnp.concatenate(
        [lap_rows.astype(jnp.int32), jnp.zeros((pad,), jnp.int32)])
    cols = jnp.concatenate(
        [lap_cols.astype(jnp.int32), jnp.zeros((pad,), jnp.int32)])
    cols2 = jnp.concatenate([cols, cols + N])
    vrep = jnp.repeat(
        jnp.concatenate([lap_vals, jnp.zeros((pad,), jnp.float32)]), LANES)

    return _sc_spmm(z2, base2, rows, cols2, vrep)
